# Initial kernel scaffold; baseline (speedup 1.0000x reference)
import jax
import jax.numpy as jnp
from jax.experimental import pallas as pl


def kernel(xs_0, xs_1, pre_words_idxs, batch_idxs_seq_lens, emb0, emb1, position_table, ln_gain, ln_bias):
    raise NotImplementedError("write your pallas kernel here")


# speedup vs baseline: 1.1300x; 1.1300x over previous
import functools

import jax
import jax.numpy as jnp
from jax import lax
from jax.experimental import pallas as pl
from jax.experimental.pallas import tpu as pltpu
from jax.experimental.pallas import tpu_sc as plsc

D = 128
MAX_LEN = 300
LN_EPS = 1e-3

NC = 2
NS = 16
LANES = 16
NW = NC * NS

CHUNK = 128
DSUB = D // LANES


def _sqrt_pos(x):
    xc = jnp.maximum(x, 1e-30)
    xi = lax.bitcast_convert_type(xc, jnp.int32)
    yi = jnp.int32(0x5F3759DF) - (xi >> 1)
    y = lax.bitcast_convert_type(yi, jnp.float32)
    for _ in range(3):
        y = y * (1.5 - 0.5 * xc * y * y)
    return xc * y


def _hsum(v):
    lane = lax.iota(jnp.int32, LANES)
    dnums = lax.GatherDimensionNumbers(
        offset_dims=(), collapsed_slice_dims=(0,), start_index_map=(0,))
    for k in (8, 4, 2, 1):
        perm = lax.bitwise_xor(lane, jnp.int32(k))
        v = v + lax.gather(
            v, perm[:, None], dnums, slice_sizes=(1,),
            mode=lax.GatherScatterMode.PROMISE_IN_BOUNDS)
    return v


def _sc_body(emb0_hbm, emb1_hbm, post_hbm, xs0_hbm, xs1_hbm, posidx_hbm,
             gain_hbm, bias_hbm,
             ann_out, cont_out, tim_out,
             idx0_v, idx1_v, idxp_v, rows0_v, rows1_v, rowsp_v,
             cont_v, ann_v, gain_v, bias_v, sem0, sem1, semp):
    wid = lax.axis_index("s") * NC + lax.axis_index("c")
    tokens_per_w = ann_out.shape[0] // NW
    nchunks = tokens_per_w // CHUNK
    base_w = wid * tokens_per_w

    pltpu.sync_copy(gain_hbm, gain_v)
    pltpu.sync_copy(bias_hbm, bias_v)

    def chunk_body(ci, _):
        base = base_w + ci * CHUNK
        pltpu.sync_copy(xs0_hbm.at[pl.ds(base, CHUNK)], idx0_v)
        pltpu.sync_copy(xs1_hbm.at[pl.ds(base, CHUNK)], idx1_v)
        pltpu.sync_copy(posidx_hbm.at[pl.ds(base, CHUNK)], idxp_v)
        cp0 = pltpu.async_copy(emb0_hbm.at[idx0_v], rows0_v, sem0)
        cp1 = pltpu.async_copy(emb1_hbm.at[idx1_v], rows1_v, sem1)
        cpp = pltpu.async_copy(post_hbm.at[idxp_v], rowsp_v, semp)
        cp0.wait()
        cp1.wait()
        cpp.wait()

        gvs = [gain_v[pl.ds(LANES * d, LANES)] for d in range(DSUB)]
        bvs = [bias_v[pl.ds(LANES * d, LANES)] for d in range(DSUB)]

        def tok(t, carry):
            avs = []
            s = jnp.zeros((LANES,), jnp.float32)
            sq = jnp.zeros((LANES,), jnp.float32)
            for d in range(DSUB):
                sl = pl.ds(LANES * d, LANES)
                c = rows0_v[t, sl] + rows1_v[t, sl]
                cont_v[t, sl] = c
                a = c + rowsp_v[t, sl]
                avs.append(a)
                s = s + a
                sq = sq + a * a
            mu = _hsum(s) * (1.0 / D)
            var = _hsum(sq) * (1.0 / D) - mu * mu
            sigma = _sqrt_pos(jnp.maximum(var, 0.0))
            r = 1.0 / (sigma + LN_EPS)
            for d in range(DSUB):
                sl = pl.ds(LANES * d, LANES)
                ann_v[t, sl] = (avs[d] - mu) * r * gvs[d] + bvs[d]
            return carry

        lax.fori_loop(0, CHUNK, tok, 0, unroll=False)

        pltpu.sync_copy(cont_v, cont_out.at[pl.ds(base, CHUNK)])
        pltpu.sync_copy(rowsp_v, tim_out.at[pl.ds(base, CHUNK)])
        pltpu.sync_copy(ann_v, ann_out.at[pl.ds(base, CHUNK)])
        return 0

    lax.fori_loop(0, nchunks, chunk_body, 0, unroll=False)


def kernel(xs_0, xs_1, pre_words_idxs, batch_idxs_seq_lens, emb0, emb1,
           position_table, ln_gain, ln_bias):
    del pre_words_idxs
    T = xs_0.shape[0]
    xs_0 = xs_0.astype(jnp.int32)
    xs_1 = xs_1.astype(jnp.int32)

    lens = batch_idxs_seq_lens.astype(jnp.int32)
    ends = jnp.cumsum(lens)
    starts = ends - lens
    g = jnp.arange(T, dtype=jnp.int32)
    seg = jnp.searchsorted(ends, g, side="right").astype(jnp.int32)
    pos_idx = ((g - starts[seg]) % MAX_LEN).astype(jnp.int32)

    out_sd = jax.ShapeDtypeStruct((T, D), jnp.float32)
    mesh = plsc.VectorSubcoreMesh(
        core_axis_name="c", subcore_axis_name="s", num_cores=NC,
        num_subcores=NS)
    run = pl.kernel(
        _sc_body,
        out_type=(out_sd, out_sd, out_sd),
        mesh=mesh,
        scratch_types=[
            pltpu.VMEM((CHUNK,), jnp.int32),
            pltpu.VMEM((CHUNK,), jnp.int32),
            pltpu.VMEM((CHUNK,), jnp.int32),
            pltpu.VMEM((CHUNK, D), jnp.float32),
            pltpu.VMEM((CHUNK, D), jnp.float32),
            pltpu.VMEM((CHUNK, D), jnp.float32),
            pltpu.VMEM((CHUNK, D), jnp.float32),
            pltpu.VMEM((CHUNK, D), jnp.float32),
            pltpu.VMEM((D,), jnp.float32),
            pltpu.VMEM((D,), jnp.float32),
            pltpu.SemaphoreType.DMA,
            pltpu.SemaphoreType.DMA,
            pltpu.SemaphoreType.DMA,
        ],
    )
    annotations, content, timing = run(
        emb0, emb1, position_table, xs_0, xs_1, pos_idx, ln_gain, ln_bias)
    return (annotations, content, timing)


# speedup vs baseline: 10.2130x; 9.0377x over previous
import functools

import jax
import jax.numpy as jnp
from jax import lax
from jax.experimental import pallas as pl
from jax.experimental.pallas import tpu as pltpu
from jax.experimental.pallas import tpu_sc as plsc

D = 128
MAX_LEN = 300
LN_EPS = 1e-3

NC = 2
NS = 16
LANES = 16
NW = NC * NS

CHUNK = 128
DSUB = D // LANES


def _sqrt_pos(x):
    xc = jnp.maximum(x, 1e-30)
    xi = lax.bitcast_convert_type(xc, jnp.int32)
    yi = jnp.int32(0x5F3759DF) - (xi >> 1)
    y = lax.bitcast_convert_type(yi, jnp.float32)
    for _ in range(3):
        y = y * (1.5 - 0.5 * xc * y * y)
    return xc * y


def _hsum(v):
    lane = lax.iota(jnp.int32, LANES)
    dnums = lax.GatherDimensionNumbers(
        offset_dims=(), collapsed_slice_dims=(0,), start_index_map=(0,))
    for k in (8, 4, 2, 1):
        perm = lax.bitwise_xor(lane, jnp.int32(k))
        v = v + lax.gather(
            v, perm[:, None], dnums, slice_sizes=(1,),
            mode=lax.GatherScatterMode.PROMISE_IN_BOUNDS)
    return v


def _sc_body(emb0_hbm, emb1_hbm, post_hbm, xs0_hbm, xs1_hbm, ends_hbm,
             gain_hbm, bias_hbm,
             ann_out, cont_out, tim_out,
             idx0_v, idx1_v, idxp_v, rows0_v, rows1_v, rowsp_v,
             cont_v, ann_v, gain_v, bias_v, ends_v, sem0, sem1, semp):
    wid = lax.axis_index("s") * NC + lax.axis_index("c")
    tokens_per_w = ann_out.shape[0] // NW
    nchunks = tokens_per_w // CHUNK
    nseg = ends_v.shape[0]
    base_w = wid * tokens_per_w

    pltpu.sync_copy(gain_hbm, gain_v)
    pltpu.sync_copy(bias_hbm, bias_v)
    pltpu.sync_copy(ends_hbm, ends_v)

    lane = lax.iota(jnp.int32, LANES)
    gsub = CHUNK // LANES
    end_vecs = [ends_v[pl.ds(b * LANES, LANES)] for b in range(nseg // LANES)]

    def chunk_body(ci, _):
        base = base_w + ci * CHUNK
        pltpu.sync_copy(xs0_hbm.at[pl.ds(base, CHUNK)], idx0_v)
        pltpu.sync_copy(xs1_hbm.at[pl.ds(base, CHUNK)], idx1_v)

        gvs = [base + i * LANES + lane for i in range(gsub)]
        starts = [jnp.zeros((LANES,), jnp.int32) for _ in range(gsub)]
        for ev in end_vecs:
            for l in range(LANES):
                e = ev[l]
                starts = [
                    jnp.maximum(st, jnp.where(e <= g, e, 0))
                    for st, g in zip(starts, gvs)]
        for i in range(gsub):
            idxp_v[pl.ds(i * LANES, LANES)] = (gvs[i] - starts[i]) % MAX_LEN

        cp0 = pltpu.async_copy(emb0_hbm.at[idx0_v], rows0_v, sem0)
        cp1 = pltpu.async_copy(emb1_hbm.at[idx1_v], rows1_v, sem1)
        cpp = pltpu.async_copy(post_hbm.at[idxp_v], rowsp_v, semp)
        cp0.wait()
        cp1.wait()
        cpp.wait()

        gvs = [gain_v[pl.ds(LANES * d, LANES)] for d in range(DSUB)]
        bvs = [bias_v[pl.ds(LANES * d, LANES)] for d in range(DSUB)]

        def tok(t, carry):
            avs = []
            s = jnp.zeros((LANES,), jnp.float32)
            sq = jnp.zeros((LANES,), jnp.float32)
            for d in range(DSUB):
                sl = pl.ds(LANES * d, LANES)
                c = rows0_v[t, sl] + rows1_v[t, sl]
                cont_v[t, sl] = c
                a = c + rowsp_v[t, sl]
                avs.append(a)
                s = s + a
                sq = sq + a * a
            mu = _hsum(s) * (1.0 / D)
            var = _hsum(sq) * (1.0 / D) - mu * mu
            sigma = _sqrt_pos(jnp.maximum(var, 0.0))
            r = 1.0 / (sigma + LN_EPS)
            for d in range(DSUB):
                sl = pl.ds(LANES * d, LANES)
                ann_v[t, sl] = (avs[d] - mu) * r * gvs[d] + bvs[d]
            return carry

        lax.fori_loop(0, CHUNK, tok, 0, unroll=False)

        pltpu.sync_copy(cont_v, cont_out.at[pl.ds(base, CHUNK)])
        pltpu.sync_copy(rowsp_v, tim_out.at[pl.ds(base, CHUNK)])
        pltpu.sync_copy(ann_v, ann_out.at[pl.ds(base, CHUNK)])
        return 0

    lax.fori_loop(0, nchunks, chunk_body, 0, unroll=False)


def kernel(xs_0, xs_1, pre_words_idxs, batch_idxs_seq_lens, emb0, emb1,
           position_table, ln_gain, ln_bias):
    del pre_words_idxs
    T = xs_0.shape[0]
    xs_0 = xs_0.astype(jnp.int32)
    xs_1 = xs_1.astype(jnp.int32)
    ends = jnp.cumsum(batch_idxs_seq_lens.astype(jnp.int32))

    out_sd = jax.ShapeDtypeStruct((T, D), jnp.float32)
    mesh = plsc.VectorSubcoreMesh(
        core_axis_name="c", subcore_axis_name="s", num_cores=NC,
        num_subcores=NS)
    run = pl.kernel(
        _sc_body,
        out_type=(out_sd, out_sd, out_sd),
        mesh=mesh,
        scratch_types=[
            pltpu.VMEM((CHUNK,), jnp.int32),
            pltpu.VMEM((CHUNK,), jnp.int32),
            pltpu.VMEM((CHUNK,), jnp.int32),
            pltpu.VMEM((CHUNK, D), jnp.float32),
            pltpu.VMEM((CHUNK, D), jnp.float32),
            pltpu.VMEM((CHUNK, D), jnp.float32),
            pltpu.VMEM((CHUNK, D), jnp.float32),
            pltpu.VMEM((CHUNK, D), jnp.float32),
            pltpu.VMEM((D,), jnp.float32),
            pltpu.VMEM((D,), jnp.float32),
            pltpu.VMEM((ends.shape[0],), jnp.int32),
            pltpu.SemaphoreType.DMA,
            pltpu.SemaphoreType.DMA,
            pltpu.SemaphoreType.DMA,
        ],
    )
    annotations, content, timing = run(
        emb0, emb1, position_table, xs_0, xs_1, ends, ln_gain, ln_bias)
    return (annotations, content, timing)
